# baseline (speedup 1.0000x reference)
import jax
import jax.numpy as jnp
from jax.experimental import pallas as pl
from jax.experimental.pallas import tpu as pltpu

IN_CH = 8
HID = 64
SUB_W = 64
GG_W = 64
NUM_SUB_LAYERS = 3
BATCH = 32
TSL = 256
NUM_CLUSTERS = BATCH * TSL
NODES_PER = 32
N = NUM_CLUSTERS * NODES_PER
NPG = TSL * NODES_PER


def _ln(x, g, b):
    mu = jnp.mean(x, axis=-1, keepdims=True)
    xc = x - mu
    var = jnp.mean(xc * xc, axis=-1, keepdims=True)
    return xc * jax.lax.rsqrt(var + 1e-5) * g + b


def _mlp_block(h, W1, b1, g1, be1, W2, b2, g2, be2, Ws, bs, gs, bes):
    o = jnp.dot(h, W1, preferred_element_type=jnp.float32) + b1
    o = jax.nn.relu(_ln(o, g1, be1))
    o = jnp.dot(o, W2, preferred_element_type=jnp.float32) + b2
    o = _ln(o, g2, be2)
    sc = _ln(jnp.dot(h, Ws, preferred_element_type=jnp.float32) + bs, gs, bes)
    return jax.nn.relu(o + sc)


def _cluster_max(h, width):
    return jnp.max(h.reshape(TSL, NODES_PER, width), axis=1)


def _body(x_ref, id_ref, mask_ref, *refs):
    out_ref = refs[-1]
    w = [r[...] for r in refs[:-1]]
    h = x_ref[...]
    i = 0
    for _ in range(NUM_SUB_LAYERS):
        h = _mlp_block(h, *w[i:i + 12])
        i += 12
        agg = _cluster_max(h, HID)
        aggb = jnp.broadcast_to(agg[:, None, :], (TSL, NODES_PER, HID))
        h = jnp.concatenate([h, aggb.reshape(NPG, HID)], axis=-1)
    Wl, bl = w[i], w[i + 1]
    i += 2
    h = jnp.dot(h, Wl, preferred_element_type=jnp.float32) + bl
    sub = _cluster_max(h, SUB_W)
    nrm = jnp.sqrt(jnp.sum(sub * sub, axis=-1, keepdims=True))
    sub = sub / jnp.maximum(nrm, 1e-12)
    ident = id_ref[...]
    Wqs, Wqi, bq, Wks, Wki, bk, Wvs, Wvi, bv = w[i:i + 9]
    q = (jnp.dot(sub, Wqs, preferred_element_type=jnp.float32)
         + jnp.dot(ident, Wqi, preferred_element_type=jnp.float32) + bq)
    k = (jnp.dot(sub, Wks, preferred_element_type=jnp.float32)
         + jnp.dot(ident, Wki, preferred_element_type=jnp.float32) + bk)
    v = (jnp.dot(sub, Wvs, preferred_element_type=jnp.float32)
         + jnp.dot(ident, Wvi, preferred_element_type=jnp.float32) + bv)
    scores = jax.lax.dot_general(q, k, (((1,), (1,)), ((), ())),
                                 preferred_element_type=jnp.float32)
    m = mask_ref[0]
    scores = jnp.where(m > 0, scores, -1e6)
    mx = jnp.max(scores, axis=-1, keepdims=True)
    e = jnp.exp(scores - mx)
    attn = e / jnp.sum(e, axis=-1, keepdims=True)
    out_ref[...] = jnp.dot(attn, v, preferred_element_type=jnp.float32)[None]


def kernel(x, cluster, edge_index, identifier, valid_len, params):
    del cluster, edge_index
    r = lambda a: a.reshape(1, -1)
    weights = []
    for p in params["sub_layers"]:
        weights += [p["W1"], r(p["b1"]), r(p["g1"]), r(p["be1"]),
                    p["W2"], r(p["b2"]), r(p["g2"]), r(p["be2"]),
                    p["Ws"], r(p["bs"]), r(p["gs"]), r(p["bes"])]
    weights += [params["Wl"], r(params["bl"])]
    for nm in ("q", "k", "v"):
        W = params["W" + nm]
        weights += [W[:SUB_W], W[SUB_W:], r(params["b" + nm])]
    mask = (jnp.arange(TSL, dtype=jnp.int32)[None, :]
            < valid_len[:, None]).astype(jnp.float32).reshape(BATCH, 1, TSL)
    in_specs = [
        pl.BlockSpec((NPG, IN_CH), lambda b: (b, 0)),
        pl.BlockSpec((TSL, 2), lambda b: (b, 0)),
        pl.BlockSpec((1, 1, TSL), lambda b: (b, 0, 0)),
    ] + [pl.BlockSpec(wt.shape, lambda b: (0, 0)) for wt in weights]
    return pl.pallas_call(
        _body,
        grid=(BATCH,),
        in_specs=in_specs,
        out_specs=pl.BlockSpec((1, TSL, GG_W), lambda b: (b, 0, 0)),
        out_shape=jax.ShapeDtypeStruct((BATCH, TSL, GG_W), jnp.float32),
        compiler_params=pltpu.CompilerParams(
            dimension_semantics=("parallel",)),
    )(x, identifier, mask, *weights)


# speedup vs baseline: 1.7934x; 1.7934x over previous
import jax
import jax.numpy as jnp
from jax.experimental import pallas as pl
from jax.experimental.pallas import tpu as pltpu

IN_CH = 8
HID = 64
SUB_W = 64
GG_W = 64
NUM_SUB_LAYERS = 3
BATCH = 32
TSL = 256
NUM_CLUSTERS = BATCH * TSL
NODES_PER = 32
N = NUM_CLUSTERS * NODES_PER
NPG = TSL * NODES_PER


def _cluster_max(h, width):
    return jnp.max(h.reshape(TSL, NODES_PER, width), axis=1)


def _tile_clusters(a, width):
    return jnp.broadcast_to(a[:, None, :], (TSL, NODES_PER, width)).reshape(
        NPG, width)


def _dot(a, b):
    return jnp.dot(a, b, preferred_element_type=jnp.float32)


def _body(x_ref, id_ref, mask_ref, *refs):
    out_ref = refs[-1]
    w = [r[...] for r in refs[:-1]]
    i = 0

    def nxt():
        nonlocal i
        i += 1
        return w[i - 1]

    ones_blk = nxt()
    ones_one = nxt()
    h = x_ref[...]
    agg = None
    for layer in range(NUM_SUB_LAYERS):
        Ah, Aa, bA, gA, beA, W2, b2, g2, be2 = (nxt() for _ in range(9))
        if layer == 0:
            y = _dot(h, Ah) + bA
        else:
            y = _dot(h, Ah) + _tile_clusters(_dot(agg, Aa) + bA, 2 * HID)
        s = _dot(y * y, ones_blk)
        z = y * jax.lax.rsqrt(s + 1e-5) * gA + beA
        o1 = jax.nn.relu(z[:, :HID])
        sc = z[:, HID:]
        y2 = _dot(o1, W2) + b2
        s2 = _dot(y2 * y2, ones_one)
        z2 = y2 * jax.lax.rsqrt(s2 + 1e-5) * g2 + be2
        h = jax.nn.relu(z2 + sc)
        agg = _cluster_max(h, HID)
    Wlh, Wla, bl = nxt(), nxt(), nxt()
    t = _dot(h, Wlh) + _tile_clusters(_dot(agg, Wla) + bl, SUB_W)
    sub = _cluster_max(t, SUB_W)
    nrm = jnp.sqrt(jnp.sum(sub * sub, axis=-1, keepdims=True))
    sub = sub / jnp.maximum(nrm, 1e-12)
    ident = id_ref[...]
    Wqs, Wqi, bq, Wks, Wki, bk, Wvs, Wvi, bv = (nxt() for _ in range(9))
    q = _dot(sub, Wqs) + _dot(ident, Wqi) + bq
    k = _dot(sub, Wks) + _dot(ident, Wki) + bk
    v = _dot(sub, Wvs) + _dot(ident, Wvi) + bv
    scores = jax.lax.dot_general(q, k, (((1,), (1,)), ((), ())),
                                 preferred_element_type=jnp.float32)
    m = mask_ref[0]
    scores = jnp.where(m > 0, scores, -1e6)
    mx = jnp.max(scores, axis=-1, keepdims=True)
    e = jnp.exp(scores - mx)
    attn = e / jnp.sum(e, axis=-1, keepdims=True)
    out_ref[...] = _dot(attn, v)[None]


def _center(W, b):
    return W - jnp.mean(W, axis=1, keepdims=True), b - jnp.mean(b)


def kernel(x, cluster, edge_index, identifier, valid_len, params):
    del cluster, edge_index
    r = lambda a: a.reshape(1, -1)
    eye2 = jnp.concatenate(
        [jnp.concatenate([jnp.ones((HID, HID)), jnp.zeros((HID, HID))], 1),
         jnp.concatenate([jnp.zeros((HID, HID)), jnp.ones((HID, HID))], 1)],
        0) / HID
    weights = [eye2.astype(jnp.float32),
               jnp.full((HID, HID), 1.0 / HID, jnp.float32)]
    for p in params["sub_layers"]:
        W1c, b1c = _center(p["W1"], p["b1"])
        Wsc, bsc = _center(p["Ws"], p["bs"])
        W2c, b2c = _center(p["W2"], p["b2"])
        A = jnp.concatenate([W1c, Wsc], axis=1)
        weights += [A[:HID], A[HID:] if A.shape[0] > HID else A[:1] * 0.0,
                    r(jnp.concatenate([b1c, bsc])),
                    r(jnp.concatenate([p["g1"], p["gs"]])),
                    r(jnp.concatenate([p["be1"], p["bes"]])),
                    W2c, r(b2c), r(p["g2"]), r(p["be2"])]
    weights += [params["Wl"][:HID], params["Wl"][HID:], r(params["bl"])]
    for nm in ("q", "k", "v"):
        W = params["W" + nm]
        weights += [W[:SUB_W], W[SUB_W:], r(params["b" + nm])]
    mask = (jnp.arange(TSL, dtype=jnp.int32)[None, :]
            < valid_len[:, None]).astype(jnp.float32).reshape(BATCH, 1, TSL)
    in_specs = [
        pl.BlockSpec((NPG, IN_CH), lambda b: (b, 0)),
        pl.BlockSpec((TSL, 2), lambda b: (b, 0)),
        pl.BlockSpec((1, 1, TSL), lambda b: (b, 0, 0)),
    ] + [pl.BlockSpec(wt.shape, lambda b: (0, 0)) for wt in weights]
    return pl.pallas_call(
        _body,
        grid=(BATCH,),
        in_specs=in_specs,
        out_specs=pl.BlockSpec((1, TSL, GG_W), lambda b: (b, 0, 0)),
        out_shape=jax.ShapeDtypeStruct((BATCH, TSL, GG_W), jnp.float32),
        compiler_params=pltpu.CompilerParams(
            dimension_semantics=("parallel",)),
    )(x, identifier, mask, *weights)
